# baseline (speedup 1.0000x reference)
import functools

import jax
import jax.numpy as jnp
from jax import lax
from jax.experimental import pallas as pl
from jax.experimental.pallas import tpu as pltpu
from jax.experimental.pallas import tpu_sc as plsc

B = 8
M = 2048
FDIM = 256
N = 8192
TILE = 512
NEG = -1000000000.0

NC, NS, L = 2, 16, 16
NW = NC * NS
RPW = (B * M) // NW
CH = 128

def _sc_body(x_hbm, ids_hbm, out_hbm, ids_v, idx_v, buf_v, zbuf_v, sem):
    wid = lax.axis_index("s") * NC + lax.axis_index("c")
    row0 = wid * RPW
    b = row0 // M
    jw = row0 % M

    pltpu.sync_copy(ids_hbm, ids_v.at[pl.ds(0, N)])
    ids_v[pl.ds(N, L)] = jnp.full((L,), B, jnp.int32)

    zero16 = jnp.zeros((L,), jnp.float32)
    for r in range(CH):
        for g in range(FDIM // L):
            zbuf_v[r, pl.ds(g * L, L)] = zero16

    def lower_bound(tgt):
        def step(_, lohi):
            lo, hi = lohi
            mid = (lo + hi) // 2
            v = ids_v[pl.ds(mid, L)][0]
            pred = v < tgt
            return jnp.where(pred, mid + 1, lo), jnp.where(pred, hi, mid)

        lo, _ = lax.fori_loop(0, 13, step, (jnp.int32(0), jnp.int32(N)))
        return lo

    start = lower_bound(b)
    count = lower_bound(b + 1) - start
    n_valid = jnp.clip(count - jw, 0, RPW)

    for c in range(RPW // CH):
        rem = jnp.clip(n_valid - c * CH, 0, CH)
        dst = row0 + c * CH

        @pl.when(rem > 0)
        def _copy_chunk():
            src = start + jw + c * CH
            lanes = lax.iota(jnp.int32, L)
            nmax = jnp.broadcast_to(jnp.int32(N - 1), (L,))
            for g in range(CH // L):
                idx_v[pl.ds(g * L, L)] = jnp.minimum(
                    lanes + jnp.broadcast_to(src + g * L, (L,)), nmax
                )
            pltpu.async_copy(x_hbm.at[idx_v], buf_v, sem).wait()

            def zrow(r, _):
                for g in range(FDIM // L):
                    buf_v[r, pl.ds(g * L, L)] = zero16
                return 0

            lax.fori_loop(rem, CH, zrow, 0)
            pltpu.sync_copy(buf_v, out_hbm.at[pl.ds(dst, CH)])

        @pl.when(rem == 0)
        def _zero_chunk():
            pltpu.sync_copy(zbuf_v, out_hbm.at[pl.ds(dst, CH)])


_sc_dense = functools.partial(
    pl.kernel,
    out_type=jax.ShapeDtypeStruct((B * M, FDIM), jnp.float32),
    mesh=plsc.VectorSubcoreMesh(
        core_axis_name="c", subcore_axis_name="s", num_cores=NC, num_subcores=NS
    ),
    scratch_types=[
        pltpu.VMEM((N + L,), jnp.int32),
        pltpu.VMEM((CH,), jnp.int32),
        pltpu.VMEM((CH, FDIM), jnp.float32),
        pltpu.VMEM((CH, FDIM), jnp.float32),
        pltpu.SemaphoreType.DMA,
    ],
)(_sc_body)


def _tc_body(ids_ref, dmask_ref, attn_ref):
    b = pl.program_id(0)
    ids = ids_ref[...]
    count = jnp.sum((ids == b).astype(jnp.int32))
    col = jax.lax.broadcasted_iota(jnp.int32, (1, M), 1)
    valid_row = col < count
    dmask_ref[0, 0, :] = valid_row[0, :]
    attn_ref[0, 0, :, :] = jnp.broadcast_to(
        jnp.where(valid_row, 0.0, NEG), (TILE, M)
    )


def kernel(x, batch_ids):
    dense_flat = _sc_dense(x, batch_ids)
    ids2d = batch_ids.reshape(64, 128)
    dmask3, attn = pl.pallas_call(
        _tc_body,
        grid=(B, M // TILE),
        in_specs=[pl.BlockSpec((64, 128), lambda b, t: (0, 0))],
        out_specs=[
            pl.BlockSpec((1, 1, M), lambda b, t: (b, 0, 0)),
            pl.BlockSpec((1, 1, TILE, M), lambda b, t: (b, 0, t, 0)),
        ],
        out_shape=[
            jax.ShapeDtypeStruct((B, 1, M), jnp.bool_),
            jax.ShapeDtypeStruct((B, 1, M, M), jnp.float32),
        ],
    )(ids2d)
    return dense_flat.reshape(B, M, FDIM), dmask3.reshape(B, M), attn


# speedup vs baseline: 1.3509x; 1.3509x over previous
import jax
import jax.numpy as jnp
from jax.experimental import pallas as pl
from jax.experimental.pallas import tpu as pltpu

B = 8
M = 2048
FDIM = 256
N = 8192
TILE = 1024
NEG = -1000000000.0


def _body(ids_ref, x_ref, dense_ref, dmask_ref, attn_ref):
    b = pl.program_id(0)
    t = pl.program_id(1)
    ids = ids_ref[...]
    count = jnp.sum((ids == b).astype(jnp.int32))
    start = jnp.sum((ids < b).astype(jnp.int32))

    jw = t * TILE
    src = jnp.minimum(start + jw, N)
    src8 = pl.multiple_of(jnp.minimum((src // 8) * 8, N - TILE - 8), 8)
    shift = src - src8
    rows = x_ref[pl.ds(src8, TILE + 8), :]
    rows = pltpu.roll(rows, (TILE + 8) - shift, 0)[:TILE, :]
    j = jw + jax.lax.broadcasted_iota(jnp.int32, (TILE, 1), 0)
    dense_ref[0, :, :] = jnp.where(j < count, rows, 0.0)

    col = jax.lax.broadcasted_iota(jnp.int32, (1, M), 1)
    valid_row = col < count
    dmask_ref[0, 0, :] = valid_row[0, :]
    attn_ref[0, 0, :, :] = jnp.broadcast_to(
        jnp.where(valid_row, 0.0, NEG), (TILE, M)
    )


def kernel(x, batch_ids):
    ids2d = batch_ids.reshape(64, 128)
    dense, dmask3, attn = pl.pallas_call(
        _body,
        grid=(B, M // TILE),
        in_specs=[
            pl.BlockSpec((64, 128), lambda b, t: (0, 0)),
            pl.BlockSpec((N, FDIM), lambda b, t: (0, 0)),
        ],
        out_specs=[
            pl.BlockSpec((1, TILE, FDIM), lambda b, t: (b, t, 0)),
            pl.BlockSpec((1, 1, M), lambda b, t: (b, 0, 0)),
            pl.BlockSpec((1, 1, TILE, M), lambda b, t: (b, 0, t, 0)),
        ],
        out_shape=[
            jax.ShapeDtypeStruct((B, M, FDIM), jnp.float32),
            jax.ShapeDtypeStruct((B, 1, M), jnp.bool_),
            jax.ShapeDtypeStruct((B, 1, M, M), jnp.float32),
        ],
    )(ids2d, x)
    return dense, dmask3.reshape(B, M), attn
